# baseline (speedup 1.0000x reference)
import dataclasses

import jax
import jax.numpy as jnp
from jax.experimental import pallas as pl
from jax.experimental.pallas import tpu as pltpu
from jax.experimental.pallas import tpu_sc as plsc

DIM = 128
WINDOW = 128
NBUF = 4
LANES = 16
NUM_CORES = 2
NUM_SUBCORES = 16


TC_ROWS = 2048
TC_FRAC_NUM, TC_FRAC_DEN = 7, 10


def _tc_onehot(idx_col):
    n_tc = idx_col.shape[0]

    def body(x_ref, o_ref):
        iota = jax.lax.broadcasted_iota(jnp.int32, (TC_ROWS, DIM), 1)
        o_ref[...] = (x_ref[...] == iota).astype(jnp.float32)

    return pl.pallas_call(
        body,
        grid=(n_tc // TC_ROWS,),
        in_specs=[pl.BlockSpec((TC_ROWS, 1), lambda i: (i, 0))],
        out_specs=pl.BlockSpec((TC_ROWS, DIM), lambda i: (i, 0)),
        out_shape=jax.ShapeDtypeStruct((n_tc, DIM), jnp.float32),
        compiler_params=pltpu.CompilerParams(dimension_semantics=("parallel",)),
    )(idx_col)


def kernel(x, I):
    B, S = x.shape
    n = B * S
    workers = NUM_CORES * NUM_SUBCORES
    align = workers * WINDOW * NBUF
    n_sc = (n - n * TC_FRAC_NUM // TC_FRAC_DEN) // align * align
    n_tc = n - n_sc
    assert n_tc % TC_ROWS == 0, (n_tc, TC_ROWS)
    chunk = n_sc // workers
    m = chunk // WINDOW
    idx = x.reshape(n).astype(jnp.int32)
    mesh = plsc.VectorSubcoreMesh(core_axis_name="core", subcore_axis_name="subcore")

    cp = pltpu.CompilerParams()
    if "needs_layout_passes" in pltpu.CompilerParams.__dataclass_fields__:
        cp = dataclasses.replace(cp, needs_layout_passes=False)

    @pl.kernel(
        out_type=jax.ShapeDtypeStruct((n_sc, DIM), I.dtype),
        mesh=mesh,
        compiler_params=cp,
        scratch_types=(
            [pltpu.VMEM((chunk,), jnp.int32)]
            + [pltpu.VMEM((WINDOW, DIM), jnp.float32)] * NBUF
            + [pltpu.SemaphoreType.DMA] * (NBUF + 1)
        ),
    )
    def onehot_kernel(table_hbm, i_hbm, o_hbm, idx_buf, *rest):
        bufs = rest[:NBUF]
        sems = rest[NBUF : 2 * NBUF]
        isem = rest[2 * NBUF]
        del table_hbm
        core = jax.lax.axis_index("core")
        sub = jax.lax.axis_index("subcore")
        wid = core * NUM_SUBCORES + sub
        base = wid * chunk

        zeros16 = jnp.zeros((LANES,), jnp.float32)
        ones16 = jnp.ones((LANES,), jnp.float32)
        lane_iota = jax.lax.iota(jnp.int32, LANES)

        pltpu.async_copy(i_hbm.at[pl.ds(base, chunk)], idx_buf, isem).wait()

        def zero_all(buf):
            @pl.loop(0, WINDOW)
            def _(r):
                row = buf.at[r]
                for c in range(0, DIM, LANES):
                    row[pl.ds(c, LANES)] = zeros16

        for b in bufs:
            zero_all(b)

        def scatter(buf, g, val):
            @pl.loop(0, WINDOW, step=LANES)
            def _(r0):
                rows = r0 + lane_iota
                cols = idx_buf[pl.ds(g * WINDOW + r0, LANES)]
                plsc.store_scatter(buf, [rows, cols], val)

        def issue(buf, g, sem):
            return pltpu.async_copy(
                buf, o_hbm.at[pl.ds(base + g * WINDOW, WINDOW)], sem
            )

        def wait(buf, g, sem):
            pltpu.make_async_copy(
                buf, o_hbm.at[pl.ds(base + g * WINDOW, WINDOW)], sem
            ).wait()

        for b in range(NBUF):
            scatter(bufs[b], b, ones16)
            issue(bufs[b], b, sems[b])

        @pl.loop(1, m // NBUF)
        def _(p):
            for b in range(NBUF):
                g = NBUF * p + b
                wait(bufs[b], g - NBUF, sems[b])
                scatter(bufs[b], g - NBUF, zeros16)
                scatter(bufs[b], g, ones16)
                issue(bufs[b], g, sems[b])

        for b in range(NBUF):
            wait(bufs[b], m - NBUF + b, sems[b])

    out_sc = onehot_kernel(I, idx[n_tc:])
    out_tc = _tc_onehot(idx[:n_tc].reshape(n_tc, 1))
    out = jnp.concatenate([out_tc, out_sc], axis=0)
    return out.reshape(B, S, DIM)


# speedup vs baseline: 4.6490x; 4.6490x over previous
import dataclasses

import jax
import jax.numpy as jnp
from jax.experimental import pallas as pl
from jax.experimental.pallas import tpu as pltpu
from jax.experimental.pallas import tpu_sc as plsc

DIM = 128
WINDOW = 256
NBUF = 2
LANES = 16
NUM_CORES = 2
NUM_SUBCORES = 16


def kernel(x, I):
    B, S = x.shape
    n = B * S
    workers = NUM_CORES * NUM_SUBCORES
    chunk = n // workers
    m = chunk // WINDOW
    idx = x.reshape(n).astype(jnp.int32)
    mesh = plsc.VectorSubcoreMesh(core_axis_name="core", subcore_axis_name="subcore")

    cp = pltpu.CompilerParams()
    if "needs_layout_passes" in pltpu.CompilerParams.__dataclass_fields__:
        cp = dataclasses.replace(cp, needs_layout_passes=False)

    @pl.kernel(
        out_type=jax.ShapeDtypeStruct((n, DIM), I.dtype),
        mesh=mesh,
        compiler_params=cp,
        scratch_types=(
            [pltpu.VMEM((chunk,), jnp.int32)]
            + [pltpu.VMEM((WINDOW, DIM), jnp.float32)] * NBUF
            + [pltpu.SemaphoreType.DMA] * (NBUF + 1)
        ),
    )
    def onehot_kernel(table_hbm, i_hbm, o_hbm, idx_buf, *rest):
        bufs = rest[:NBUF]
        sems = rest[NBUF : 2 * NBUF]
        isem = rest[2 * NBUF]
        del table_hbm
        core = jax.lax.axis_index("core")
        sub = jax.lax.axis_index("subcore")
        wid = core * NUM_SUBCORES + sub
        base = wid * chunk

        zeros16 = jnp.zeros((LANES,), jnp.float32)
        ones16 = jnp.ones((LANES,), jnp.float32)
        lane_iota = jax.lax.iota(jnp.int32, LANES)

        pltpu.async_copy(i_hbm.at[pl.ds(base, chunk)], idx_buf, isem).wait()

        def zero_all(buf):
            @pl.loop(0, WINDOW)
            def _(r):
                row = buf.at[r]
                for c in range(0, DIM, LANES):
                    row[pl.ds(c, LANES)] = zeros16

        def scatter(buf, g, val):
            @pl.loop(0, WINDOW, step=LANES)
            def _(r0):
                rows = r0 + lane_iota
                cols = idx_buf[pl.ds(g * WINDOW + r0, LANES)]
                plsc.store_scatter(buf, [rows, cols], val)

        def issue(buf, g, sem):
            return pltpu.async_copy(
                buf, o_hbm.at[pl.ds(base + g * WINDOW, WINDOW)], sem
            )

        def wait(buf, g, sem):
            pltpu.make_async_copy(
                buf, o_hbm.at[pl.ds(base + g * WINDOW, WINDOW)], sem
            ).wait()

        for b in range(NBUF):
            zero_all(bufs[b])
            scatter(bufs[b], b, ones16)
            issue(bufs[b], b, sems[b])

        @pl.loop(1, m // NBUF)
        def _(p):
            for b in range(NBUF):
                g = NBUF * p + b
                wait(bufs[b], g - NBUF, sems[b])
                scatter(bufs[b], g - NBUF, zeros16)
                scatter(bufs[b], g, ones16)
                issue(bufs[b], g, sems[b])

        for b in range(NBUF):
            wait(bufs[b], m - NBUF + b, sems[b])

    out = onehot_kernel(I, idx)
    return out.reshape(B, S, DIM)
